# Initial kernel scaffold; baseline (speedup 1.0000x reference)
import jax
import jax.numpy as jnp
from jax.experimental import pallas as pl


def kernel(hidden_states, gate_w, gate_proj_w, up_proj_w, down_proj_w):
    raise NotImplementedError("write your pallas kernel here")


# speedup vs baseline: 1.6290x; 1.6290x over previous
import functools

import jax
import jax.numpy as jnp
from jax.experimental import pallas as pl
from jax.experimental.pallas import tpu as pltpu

E = 8
TOP_K = 2
BM = 512


def _moe_body(x_ref, wg_ref, wu_ref, wd_ref, rw_ref, out_ref):
    e = pl.program_id(1)
    x = x_ref[...]
    g = jax.lax.dot_general(x, wg_ref[0], (((1,), (1,)), ((), ())),
                            preferred_element_type=jnp.float32)
    u = jax.lax.dot_general(x, wu_ref[0], (((1,), (1,)), ((), ())),
                            preferred_element_type=jnp.float32)
    a = g * jax.lax.logistic(g) * u
    y = jax.lax.dot_general(a, wd_ref[0], (((1,), (1,)), ((), ())),
                            preferred_element_type=jnp.float32)
    lane = jax.lax.broadcasted_iota(jnp.int32, (BM, E), 1)
    w_col = jnp.sum(jnp.where(lane == e, rw_ref[...], 0.0), axis=1,
                    keepdims=True)

    @pl.when(e == 0)
    def _():
        out_ref[...] = jnp.zeros_like(out_ref)

    out_ref[...] += w_col * y


def kernel(hidden_states, gate_w, gate_proj_w, up_proj_w, down_proj_w):
    b, s, h = hidden_states.shape
    x = hidden_states.reshape(-1, h)
    t = x.shape[0]
    i_dim = gate_proj_w.shape[1]

    router_logits = x @ gate_w.T
    probs = jax.nn.softmax(router_logits, axis=-1)
    top_vals, top_idx = jax.lax.top_k(probs, TOP_K)
    top_vals = top_vals / jnp.sum(top_vals, axis=-1, keepdims=True)
    rows = jnp.arange(t)[:, None]
    routing_weights = jnp.zeros((t, E), dtype=x.dtype).at[rows, top_idx].set(top_vals)

    nb = t // BM
    out = pl.pallas_call(
        _moe_body,
        grid=(nb, E),
        in_specs=[
            pl.BlockSpec((BM, h), lambda b_, e_: (b_, 0)),
            pl.BlockSpec((1, i_dim, h), lambda b_, e_: (e_, 0, 0)),
            pl.BlockSpec((1, i_dim, h), lambda b_, e_: (e_, 0, 0)),
            pl.BlockSpec((1, h, i_dim), lambda b_, e_: (e_, 0, 0)),
            pl.BlockSpec((BM, E), lambda b_, e_: (b_, 0)),
        ],
        out_specs=pl.BlockSpec((BM, h), lambda b_, e_: (b_, 0)),
        out_shape=jax.ShapeDtypeStruct((t, h), jnp.float32),
    )(x, gate_proj_w, up_proj_w, down_proj_w, routing_weights)
    return out.reshape(b, s, h)


# speedup vs baseline: 2.2027x; 1.3522x over previous
import functools

import jax
import jax.numpy as jnp
from jax import lax
from jax.experimental import pallas as pl
from jax.experimental.pallas import tpu as pltpu
from jax.experimental.pallas import tpu_sc as plsc

NE = 8
KSEL = 2
BM = 256
NWORK = 32
CH = 32


def _mlp_body(be_ref, x_ref, wg_ref, wu_ref, wd_ref, w_ref, y_ref):
    x = x_ref[...]
    g = jax.lax.dot_general(x, wg_ref[0], (((1,), (1,)), ((), ())),
                            preferred_element_type=jnp.float32)
    u = jax.lax.dot_general(x, wu_ref[0], (((1,), (1,)), ((), ())),
                            preferred_element_type=jnp.float32)
    a = g * jax.lax.logistic(g) * u
    y = jax.lax.dot_general(a, wd_ref[0], (((1,), (1,)), ((), ())),
                            preferred_element_type=jnp.float32)
    y_ref[...] = y * w_ref[:, :1]


def _dispatch_body(x_hbm, idx0_hbm, idx1_hbm, slot_hbm, w16_hbm,
                   xs_out, ws_out, bufx, bufw, idxv, idxw, sem):
    wid = lax.axis_index("s") * 2 + lax.axis_index("c")
    base = wid * (4096 // NWORK)
    for c in range(4096 // NWORK // CH):
        tb = base + c * CH
        pltpu.sync_copy(x_hbm.at[pl.ds(tb, CH)], bufx)
        pltpu.sync_copy(idx0_hbm.at[pl.ds(tb, CH)], idxv)
        pltpu.async_copy(bufx, xs_out.at[idxv], sem).wait()
        pltpu.sync_copy(idx1_hbm.at[pl.ds(tb, CH)], idxv)
        pltpu.async_copy(bufx, xs_out.at[idxv], sem).wait()
        ab = 2 * tb
        pltpu.sync_copy(w16_hbm.at[pl.ds(ab, 2 * CH)], bufw)
        pltpu.sync_copy(slot_hbm.at[pl.ds(ab, 2 * CH)], idxw)
        pltpu.async_copy(bufw, ws_out.at[idxw], sem).wait()


CH4 = 16


def _combine_body(y_hbm, idx0_hbm, idx1_hbm, out_hbm, buf0, buf1, idxv, sem):
    wid = lax.axis_index("s") * 2 + lax.axis_index("c")
    base = wid * (4096 // NWORK)
    h = buf0.shape[1]

    def chunk(c, carry):
        tb = base + c * CH4
        pltpu.sync_copy(idx0_hbm.at[pl.ds(tb, CH4)], idxv)
        pltpu.async_copy(y_hbm.at[idxv], buf0, sem).wait()
        pltpu.sync_copy(idx1_hbm.at[pl.ds(tb, CH4)], idxv)
        pltpu.async_copy(y_hbm.at[idxv], buf1, sem).wait()

        def row(r, carry2):
            for j in range(h // 16):
                buf0[r, pl.ds(j * 16, 16)] += buf1[r, pl.ds(j * 16, 16)]
            return carry2

        lax.fori_loop(0, CH4, row, 0)
        pltpu.sync_copy(buf0, out_hbm.at[pl.ds(tb, CH4)])
        return carry

    lax.fori_loop(0, 4096 // NWORK // CH4, chunk, 0)


def kernel(hidden_states, gate_w, gate_proj_w, up_proj_w, down_proj_w):
    b, s, h = hidden_states.shape
    x = hidden_states.reshape(-1, h)
    t = x.shape[0]
    i_dim = gate_proj_w.shape[1]
    a_tot = t * KSEL
    pad_t = a_tot + NE * BM
    nb = pad_t // BM

    router_logits = x @ gate_w.T
    probs = jax.nn.softmax(router_logits, axis=-1)
    top_vals, top_idx = jax.lax.top_k(probs, KSEL)
    top_vals = top_vals / jnp.sum(top_vals, axis=-1, keepdims=True)

    e_flat = top_idx.reshape(-1)
    onehot = (e_flat[:, None] == jnp.arange(NE)[None, :]).astype(jnp.int32)
    ranks_inc = jnp.cumsum(onehot, axis=0)
    rank = jnp.sum(ranks_inc * onehot, axis=1) - 1
    counts = ranks_inc[-1]
    padded = ((counts + BM - 1) // BM) * BM
    cpad = jnp.cumsum(padded)
    pad_off = cpad - padded
    slot = (jnp.sum(onehot * pad_off[None, :], axis=1) + rank).astype(jnp.int32)
    idx0 = slot[0::2]
    idx1 = slot[1::2]
    block_expert = jnp.clip(
        jnp.searchsorted(cpad, jnp.arange(nb) * BM, side="right"), 0, NE - 1
    ).astype(jnp.int32)
    w16 = jnp.broadcast_to(top_vals.reshape(-1)[:, None], (a_tot, 128))

    mesh = plsc.VectorSubcoreMesh(core_axis_name="c", subcore_axis_name="s")
    x_sorted, w_slot = pl.kernel(
        _dispatch_body,
        out_type=[
            jax.ShapeDtypeStruct((pad_t, h), jnp.float32),
            jax.ShapeDtypeStruct((pad_t, 128), jnp.float32),
        ],
        mesh=mesh,
        scratch_types=[
            pltpu.VMEM((CH, h), jnp.float32),
            pltpu.VMEM((2 * CH, 128), jnp.float32),
            pltpu.VMEM((CH,), jnp.int32),
            pltpu.VMEM((2 * CH,), jnp.int32),
            pltpu.SemaphoreType.DMA,
        ],
    )(x, idx0, idx1, slot, w16)

    grid_spec = pltpu.PrefetchScalarGridSpec(
        num_scalar_prefetch=1,
        grid=(nb,),
        in_specs=[
            pl.BlockSpec((BM, h), lambda bi, be: (bi, 0)),
            pl.BlockSpec((1, i_dim, h), lambda bi, be: (be[bi], 0, 0)),
            pl.BlockSpec((1, i_dim, h), lambda bi, be: (be[bi], 0, 0)),
            pl.BlockSpec((1, h, i_dim), lambda bi, be: (be[bi], 0, 0)),
            pl.BlockSpec((BM, 128), lambda bi, be: (bi, 0)),
        ],
        out_specs=pl.BlockSpec((BM, h), lambda bi, be: (bi, 0)),
    )
    y_sorted = pl.pallas_call(
        _mlp_body,
        grid_spec=grid_spec,
        out_shape=jax.ShapeDtypeStruct((pad_t, h), jnp.float32),
    )(block_expert, x_sorted, gate_proj_w, up_proj_w, down_proj_w, w_slot)

    out = pl.kernel(
        _combine_body,
        out_type=jax.ShapeDtypeStruct((t, h), jnp.float32),
        mesh=mesh,
        scratch_types=[
            pltpu.VMEM((CH4, h), jnp.float32),
            pltpu.VMEM((CH4, h), jnp.float32),
            pltpu.VMEM((CH4,), jnp.int32),
            pltpu.SemaphoreType.DMA,
        ],
    )(y_sorted, idx0, idx1)
    return out.reshape(b, s, h)
